# baseline (speedup 1.0000x reference)
import functools

import jax
import jax.numpy as jnp
from jax import lax
from jax.experimental import pallas as pl
from jax.experimental.pallas import tpu as pltpu
from jax.experimental.pallas import tpu_sc as plsc

_NC = 2
_NS = 16
_NW = _NC * _NS
_L = 16


def _matvec_body(tab_ref, w_ref, b_ref, t_ref):
    x = tab_ref[...]
    w = w_ref[...]
    t_ref[...] = jnp.sum(x * w, axis=0, keepdims=True) + b_ref[0]


def _tc_matvec(tableT, WT, b, blk):
    D, V = tableT.shape
    grid = pl.cdiv(V, blk)
    return pl.pallas_call(
        _matvec_body,
        grid=(grid,),
        in_specs=[
            pl.BlockSpec((D, blk), lambda i: (0, i)),
            pl.BlockSpec((D, 1), lambda i: (0, 0)),
            pl.BlockSpec((1,), lambda i: (0,)),
        ],
        out_specs=pl.BlockSpec((1, blk), lambda i: (0, i)),
        out_shape=jax.ShapeDtypeStruct((1, V), jnp.float32),
    )(tableT, WT, b)


def _sc_pool_sigmoid(t, ids_wsj, B, S):
    ids_per_w = (B * S) // _NW
    rows_per_w = B // _NW
    n_acc = rows_per_w // _L
    mesh = plsc.VectorSubcoreMesh(core_axis_name="c", subcore_axis_name="s")

    @functools.partial(
        pl.kernel,
        out_type=jax.ShapeDtypeStruct((B,), jnp.float32),
        mesh=mesh,
        scratch_types=[
            pltpu.VMEM((ids_per_w,), jnp.int32),
            pltpu.VMEM((ids_per_w,), jnp.float32),
            pltpu.VMEM((rows_per_w,), jnp.float32),
            pltpu.SemaphoreType.DMA,
        ],
    )
    def sc_k(t_hbm, ids_hbm, out_hbm, idx_v, vals_v, out_v, sem):
        wid = lax.axis_index("s") * _NC + lax.axis_index("c")
        base = wid * ids_per_w
        pltpu.sync_copy(ids_hbm.at[pl.ds(base, ids_per_w)], idx_v)
        pltpu.async_copy(t_hbm.at[idx_v], vals_v, sem).wait()
        inv = jnp.float32(1.0 / S)

        def body(s, accs):
            off = s * rows_per_w
            return tuple(
                accs[i] + vals_v[pl.ds(off + i * _L, _L)] for i in range(n_acc)
            )

        accs = lax.fori_loop(
            0, S, body, tuple(jnp.zeros((_L,), jnp.float32) for _ in range(n_acc))
        )
        for i in range(n_acc):
            y = accs[i] * inv
            out_v[pl.ds(i * _L, _L)] = 1.0 / (1.0 + jnp.exp(-y))
        pltpu.sync_copy(out_v, out_hbm.at[pl.ds(wid * rows_per_w, rows_per_w)])

    return sc_k(t, ids_wsj)


def kernel(ids, table, W, b):
    B, S = ids.shape
    V, D = table.shape
    t = _tc_matvec(table.T, W.T, b, blk=65536)
    rows_per_w = B // _NW
    ids_wsj = ids.reshape(_NW, rows_per_w, S).transpose(0, 2, 1).reshape(B * S)
    out = _sc_pool_sigmoid(t.reshape(V), ids_wsj, B, S)
    return out.reshape(B, 1)


# speedup vs baseline: 1.0632x; 1.0632x over previous
import functools

import jax
import jax.numpy as jnp
from jax import lax
from jax.experimental import pallas as pl
from jax.experimental.pallas import tpu as pltpu
from jax.experimental.pallas import tpu_sc as plsc

_NC = 2
_NS = 16
_NW = _NC * _NS
_L = 16


def _matvec_body(tab_ref, w_ref, b_ref, t_ref):
    x = tab_ref[...]
    w = w_ref[...]
    t_ref[...] = jnp.sum(x * w, axis=0, keepdims=True) + b_ref[0]


def _tc_matvec(tableT, WT, b, blk):
    D, V = tableT.shape
    grid = pl.cdiv(V, blk)
    return pl.pallas_call(
        _matvec_body,
        grid=(grid,),
        in_specs=[
            pl.BlockSpec((D, blk), lambda i: (0, i)),
            pl.BlockSpec((D, 1), lambda i: (0, 0)),
            pl.BlockSpec((1,), lambda i: (0,)),
        ],
        out_specs=pl.BlockSpec((1, blk), lambda i: (0, i)),
        out_shape=jax.ShapeDtypeStruct((1, V), jnp.float32),
    )(tableT, WT, b)


def _sc_pool_sigmoid(t, ids_wsj, B, S):
    V = t.shape[0]
    ids_per_w = (B * S) // _NW
    rows_per_w = B // _NW
    n_acc = rows_per_w // _L
    stage_workers = 8
    stage_chunk = V // stage_workers
    mesh = plsc.VectorSubcoreMesh(core_axis_name="c", subcore_axis_name="s")

    @functools.partial(
        pl.kernel,
        out_type=jax.ShapeDtypeStruct((B,), jnp.float32),
        mesh=mesh,
        scratch_types=[
            pltpu.VMEM((ids_per_w,), jnp.int32),
            pltpu.VMEM((ids_per_w,), jnp.float32),
            pltpu.VMEM((rows_per_w,), jnp.float32),
            pltpu.VMEM_SHARED((V,), jnp.float32),
            pltpu.SemaphoreType.DMA,
            pltpu.SemaphoreType.DMA,
        ],
    )
    def sc_k(t_hbm, ids_hbm, out_hbm, idx_v, vals_v, out_v, t_sh, sem, sem2):
        cid = lax.axis_index("c")
        sid = lax.axis_index("s")
        wid = sid * _NC + cid
        base = wid * ids_per_w
        @pl.when(sid < stage_workers)
        def _():
            off = sid * stage_chunk
            n_sub = 5
            sub = stage_chunk // n_sub

            def stage(k, _):
                o = off + k * sub
                pltpu.sync_copy(t_hbm.at[pl.ds(o, sub)], vals_v.at[pl.ds(0, sub)])
                pltpu.sync_copy(vals_v.at[pl.ds(0, sub)], t_sh.at[pl.ds(o, sub)])
                return 0

            lax.fori_loop(0, n_sub, stage, 0)

        pltpu.sync_copy(ids_hbm.at[pl.ds(base, ids_per_w)], idx_v)
        plsc.subcore_barrier()
        pltpu.async_copy(t_sh.at[idx_v], vals_v, sem).wait()
        inv = jnp.float32(1.0 / S)

        def body(s, accs):
            off = s * rows_per_w
            return tuple(
                accs[i] + vals_v[pl.ds(off + i * _L, _L)] for i in range(n_acc)
            )

        accs = lax.fori_loop(
            0, S, body, tuple(jnp.zeros((_L,), jnp.float32) for _ in range(n_acc))
        )
        for i in range(n_acc):
            y = accs[i] * inv
            out_v[pl.ds(i * _L, _L)] = 1.0 / (1.0 + jnp.exp(-y))
        pltpu.sync_copy(out_v, out_hbm.at[pl.ds(wid * rows_per_w, rows_per_w)])

    return sc_k(t, ids_wsj)


def kernel(ids, table, W, b):
    B, S = ids.shape
    V, D = table.shape
    t = _tc_matvec(table.T, W.T, b, blk=32768)
    rows_per_w = B // _NW
    ids_wsj = ids.reshape(_NW, rows_per_w, S).transpose(0, 2, 1).reshape(B * S)
    out = _sc_pool_sigmoid(t.reshape(V), ids_wsj, B, S)
    return out.reshape(B, 1)
